# Initial kernel scaffold; baseline (speedup 1.0000x reference)
import jax
import jax.numpy as jnp
from jax.experimental import pallas as pl


def kernel(x, edge_index, edge_attr, fragments_edge_index, batch, fragment_types, params):
    raise NotImplementedError("write your pallas kernel here")


# speedup vs baseline: 2.4189x; 2.4189x over previous
import functools

import jax
import jax.numpy as jnp
from jax import lax
from jax.experimental import pallas as pl
from jax.experimental.pallas import tpu as pltpu
from jax.experimental.pallas import tpu_sc as plsc

_NC = 2
_NS = 16
_NW = _NC * _NS
_CH = 128
_BLK = 4096
_EPS = 1e-5


def _rup(v, m):
    return (v + m - 1) // m * m


def _sc_gather(table, idx_pad):
    K = idx_pad.shape[0]
    W = table.shape[1]
    per_w = K // _NW
    n_it = per_w // _CH
    mesh = plsc.VectorSubcoreMesh(core_axis_name="c", subcore_axis_name="s")

    @functools.partial(
        pl.kernel,
        out_type=jax.ShapeDtypeStruct((K, W), table.dtype),
        mesh=mesh,
        compiler_params=pltpu.CompilerParams(use_tc_tiling_on_sc=False),
        scratch_types=[
            pltpu.VMEM((_CH,), jnp.int32),
            pltpu.VMEM((_CH, W), table.dtype),
            pltpu.SemaphoreType.DMA,
        ],
    )
    def k(table_hbm, idx_hbm, out_hbm, idx_v, rows_v, sem):
        wid = lax.axis_index("s") * _NC + lax.axis_index("c")
        base = wid * per_w

        def body(i, carry):
            off = base + i * _CH
            pltpu.sync_copy(idx_hbm.at[pl.ds(off, _CH)], idx_v)
            pltpu.async_copy(table_hbm.at[idx_v], rows_v, sem).wait()
            pltpu.sync_copy(rows_v, out_hbm.at[pl.ds(off, _CH)])
            return carry

        lax.fori_loop(0, n_it, body, 0)

    return k(table, idx_pad)


def _zero_region(zsrc, dst, row0, rows):
    zr = zsrc.shape[0]
    full, rem = rows // zr, rows % zr
    for j in range(full):
        pltpu.sync_copy(zsrc.at[pl.ds(0, zr)], dst.at[pl.ds(row0 + j * zr, zr)])
    if rem:
        pltpu.sync_copy(zsrc.at[pl.ds(0, rem)],
                        dst.at[pl.ds(row0 + full * zr, rem)])


def _sc_scatter(data, idx_pad, num_out_pad, width=64):
    K = idx_pad.shape[0]
    H = num_out_pad // 2
    per_t = K // _NS
    n_it = per_t // _CH
    ptr = H // _NS
    has_data = data is not None
    mesh = plsc.VectorSubcoreMesh(core_axis_name="c", subcore_axis_name="s")

    if has_data:
        out_type = (jax.ShapeDtypeStruct((2 * H, width), jnp.float32),
                    jax.ShapeDtypeStruct((2 * H, 8), jnp.float32))
        scratch = [
            pltpu.VMEM((_CH,), jnp.int32),
            pltpu.VMEM((_CH,), jnp.int32),
            pltpu.VMEM((_CH, width), jnp.float32),
            pltpu.VMEM((_CH, 8), jnp.float32),
            pltpu.VMEM_SHARED((H + 16, width), jnp.float32),
            pltpu.VMEM_SHARED((H + 16, 8), jnp.float32),
        ]
    else:
        out_type = jax.ShapeDtypeStruct((2 * H,), jnp.float32)
        scratch = [
            pltpu.VMEM((_CH,), jnp.int32),
            pltpu.VMEM((_CH,), jnp.int32),
            pltpu.VMEM((_CH,), jnp.float32),
            pltpu.VMEM_SHARED((H + 16,), jnp.float32),
        ]

    @functools.partial(pl.kernel, out_type=out_type, mesh=mesh,
                       compiler_params=pltpu.CompilerParams(
                           use_tc_tiling_on_sc=False),
                       scratch_types=scratch)
    def k(*refs):
        if has_data:
            (data_hbm, idx_hbm, ones_hbm, zw_hbm, z8_hbm,
             sums_hbm, cnt_hbm,
             idx_v, loc_v, data_v, ones_v, sp_sum, sp_cnt) = refs
        else:
            (idx_hbm, ones_hbm, z1_hbm,
             cnt_hbm,
             idx_v, loc_v, ones_v, sp_cnt) = refs
        c = lax.axis_index("c")
        s = lax.axis_index("s")
        cbase = c * H

        pltpu.sync_copy(ones_hbm, ones_v)
        if has_data:
            _zero_region(zw_hbm, sp_sum, s * ptr, ptr)
            _zero_region(z8_hbm, sp_cnt, s * ptr, ptr)
        else:
            _zero_region(z1_hbm, sp_cnt, s * ptr, ptr)
        plsc.subcore_barrier()

        base = s * per_t

        def body(i, carry):
            off = base + i * _CH
            pltpu.sync_copy(idx_hbm.at[pl.ds(off, _CH)], idx_v)
            if has_data:
                pltpu.sync_copy(data_hbm.at[pl.ds(off, _CH)], data_v)
            for k8 in range(_CH // 16):
                iv = idx_v[pl.ds(k8 * 16, 16)]
                loc = iv - cbase
                inb = (iv >= cbase) & (iv < cbase + H)
                loc_v[pl.ds(k8 * 16, 16)] = jnp.where(inb, loc, H)
            if has_data:
                pltpu.sync_copy(data_v, sp_sum.at[loc_v], add=True)
            pltpu.sync_copy(ones_v, sp_cnt.at[loc_v], add=True)
            return carry

        lax.fori_loop(0, n_it, body, 0)
        plsc.subcore_barrier()

        r0 = s * ptr
        if has_data:
            pltpu.sync_copy(sp_sum.at[pl.ds(r0, ptr)],
                            sums_hbm.at[pl.ds(cbase + r0, ptr)])
        pltpu.sync_copy(sp_cnt.at[pl.ds(r0, ptr)],
                        cnt_hbm.at[pl.ds(cbase + r0, ptr)])

    if has_data:
        ones = jnp.ones((_CH, 8), jnp.float32)
        zw = jnp.zeros((2048, width), jnp.float32)
        z8 = jnp.zeros((2048, 8), jnp.float32)
        return k(data, idx_pad, ones, zw, z8)
    ones = jnp.ones((_CH,), jnp.float32)
    z1 = jnp.zeros((2048,), jnp.float32)
    return k(idx_pad, ones, z1)


_USE_SC_GATHER = True
_USE_SC_SCATTER = True
_USE_JNP_EMBED = False


def _gather(table, idx_pad):
    if _USE_SC_GATHER:
        return _sc_gather(table, idx_pad)
    return table[jnp.minimum(idx_pad, table.shape[0] - 1)]


def _scatter(data, idx_pad, num_out_pad, width=64):
    if _USE_SC_SCATTER:
        return _sc_scatter(data, idx_pad, num_out_pad, width)
    if data is None:
        return jax.ops.segment_sum(jnp.ones(idx_pad.shape, jnp.float32),
                                   idx_pad, num_segments=num_out_pad + 1)[:num_out_pad]
    sums = jax.ops.segment_sum(data, idx_pad,
                               num_segments=num_out_pad + 1)[:num_out_pad]
    cnt = jax.ops.segment_sum(jnp.ones(idx_pad.shape, jnp.float32), idx_pad,
                              num_segments=num_out_pad + 1)[:num_out_pad]
    return sums, jnp.tile(cnt[:, None], (1, 8))


def _tc_stage(parts, Wmat, bias, R, want_stats=True, blk=_BLK, pre_add=False):
    Rpad = parts[0]["x"].shape[0]
    nb = Rpad // blk
    has_w = Wmat is not None
    has_b = bias is not None
    dout = Wmat.shape[1] if has_w else parts[0]["x"].shape[1]
    cfg = tuple((p["x"].shape[1], p.get("stats") is not None, p.get("relu", False),
                 p.get("cnt") is not None, p.get("scale", 1.0)) for p in parts)
    need_mask = R < Rpad

    def body(*refs):
        it = iter(refs)
        i = pl.program_id(0)
        xs = []
        for (d, has_s, rl, has_c, scl) in cfg:
            xr = next(it)
            st = next(it) if has_s else None
            cr = next(it) if has_c else None
            x = xr[...]
            if has_s:
                sv = st[...]
                sr = p.get("stats_R", R)
                d0 = sv[0:1, :] * (1.0 / sr)
                m = sv[2:3, :] + d0
                var = jnp.maximum(sv[1:2, :] * (1.0 / sr) - d0 * d0, 0.0)
                x = (x - m) / jnp.sqrt(var + _EPS)
            if rl:
                x = jnp.maximum(x, 0.0)
            if has_c:
                cnt = cr[...][:, 0:1]
                x = x / jnp.maximum(cnt, 1.0)
            if scl != 1.0:
                x = x * scl
            xs.append(x)
        if has_w:
            wr = next(it)
            if pre_add:
                x = xs[0]
                for xo in xs[1:]:
                    x = x + xo
                acc = jnp.dot(x, wr[...],
                              preferred_element_type=jnp.float32)
            else:
                acc = None
                woff = 0
                for x, (d, *_rest) in zip(xs, cfg):
                    t = jnp.dot(x, wr[pl.ds(woff, d), :],
                                preferred_element_type=jnp.float32)
                    acc = t if acc is None else acc + t
                    woff += d
        else:
            acc = xs[0]
            for x in xs[1:]:
                acc = acc + x
        if has_b:
            acc = acc + next(it)[...][0:1, :]
        if need_mask:
            rid = i * blk + lax.broadcasted_iota(jnp.int32, (blk, 1), 0)
            acc = jnp.where(rid < R, acc, 0.0)
        z_ref = next(it)
        z_ref[...] = acc
        if want_stats:
            st_ref = next(it)

            @pl.when(i == 0)
            def _():
                st_ref[...] = jnp.zeros((8, dout), jnp.float32)
                st_ref[2:3, :] = jnp.sum(acc, axis=0, keepdims=True) * (1.0 / blk)

            cshift = st_ref[2:3, :]
            d = acc - cshift
            if need_mask:
                rid2 = i * blk + lax.broadcasted_iota(jnp.int32, (blk, 1), 0)
                d = jnp.where(rid2 < R, d, 0.0)
            st_ref[0:1, :] += jnp.sum(d, axis=0, keepdims=True)
            st_ref[1:2, :] += jnp.sum(d * d, axis=0, keepdims=True)

    in_specs = []
    args = []
    for p, (d, has_s, rl, has_c, scl) in zip(parts, cfg):
        in_specs.append(pl.BlockSpec((blk, d), lambda i: (i, 0)))
        args.append(p["x"])
        if has_s:
            in_specs.append(pl.BlockSpec((8, d), lambda i: (0, 0)))
            args.append(p["stats"])
        if has_c:
            in_specs.append(pl.BlockSpec((blk, 8), lambda i: (i, 0)))
            args.append(p["cnt"])
    if has_w:
        dtot = Wmat.shape[0]
        in_specs.append(pl.BlockSpec((dtot, dout), lambda i: (0, 0)))
        args.append(Wmat)
    if has_b:
        in_specs.append(pl.BlockSpec((8, dout), lambda i: (0, 0)))
        args.append(jnp.broadcast_to(bias[None, :], (8, dout)))

    out_shape = [jax.ShapeDtypeStruct((Rpad, dout), jnp.float32)]
    out_specs = [pl.BlockSpec((blk, dout), lambda i: (i, 0))]
    if want_stats:
        out_shape.append(jax.ShapeDtypeStruct((8, dout), jnp.float32))
        out_specs.append(pl.BlockSpec((8, dout), lambda i: (0, 0)))

    res = pl.pallas_call(
        body, grid=(nb,), in_specs=in_specs, out_specs=out_specs,
        out_shape=out_shape,
    )(*args)
    return (res[0], res[1]) if want_stats else res[0]


def _tc_embed_atoms(xp, tables, R):
    Rpad = xp.shape[0]
    nb = Rpad // _BLK
    need_mask = R < Rpad

    def body(x_ref, t_ref, o_ref):
        i = pl.program_id(0)
        x = x_ref[...]
        acc = jnp.zeros((_BLK, 64), jnp.float32)
        iot = lax.broadcasted_iota(jnp.int32, (1, 128), 1)
        for f in range(9):
            oh = (x[:, f:f + 1] == iot).astype(jnp.float32)
            acc = acc + jnp.dot(oh, t_ref[f], preferred_element_type=jnp.float32, precision=lax.Precision.HIGHEST)
        if need_mask:
            rid = i * _BLK + lax.broadcasted_iota(jnp.int32, (_BLK, 1), 0)
            acc = jnp.where(rid < R, acc, 0.0)
        o_ref[...] = acc

    return pl.pallas_call(
        body, grid=(nb,),
        in_specs=[pl.BlockSpec((_BLK, 16), lambda i: (i, 0)),
                  pl.BlockSpec((9, 128, 64), lambda i: (0, 0, 0))],
        out_specs=pl.BlockSpec((_BLK, 64), lambda i: (i, 0)),
        out_shape=jax.ShapeDtypeStruct((Rpad, 64), jnp.float32),
    )(xp, tables)


def _tc_embed_edges(eap, tables, R):
    Rpad = eap.shape[0]
    nb = Rpad // _BLK
    need_mask = R < Rpad

    def body(x_ref, t_ref, o_ref):
        i = pl.program_id(0)
        x = x_ref[...]
        acc = jnp.zeros((_BLK, 64), jnp.float32)
        iot = lax.broadcasted_iota(jnp.int32, (1, 32), 1)
        for f in range(3):
            oh = (x[:, f:f + 1] == iot).astype(jnp.float32)
            acc = acc + jnp.dot(oh, t_ref[f], preferred_element_type=jnp.float32, precision=lax.Precision.HIGHEST)
        if need_mask:
            rid = i * _BLK + lax.broadcasted_iota(jnp.int32, (_BLK, 1), 0)
            acc = jnp.where(rid < R, acc, 0.0)
        o_ref[...] = acc

    return pl.pallas_call(
        body, grid=(nb,),
        in_specs=[pl.BlockSpec((_BLK, 8), lambda i: (i, 0)),
                  pl.BlockSpec((3, 32, 64), lambda i: (0, 0, 0))],
        out_specs=pl.BlockSpec((_BLK, 64), lambda i: (i, 0)),
        out_shape=jax.ShapeDtypeStruct((Rpad, 64), jnp.float32),
    )(eap, tables)


def _tc_embed_frags(ftp, table, svec, R):
    Rpad = ftp.shape[0]
    nb = Rpad // _BLK if Rpad >= _BLK else 1
    blk = Rpad // nb
    need_mask = R < Rpad

    def body(x_ref, t_ref, s_ref, o_ref):
        i = pl.program_id(0)
        x = x_ref[...]
        iot = lax.broadcasted_iota(jnp.int32, (1, 32), 1)
        oh = (x[:, 0:1] == iot).astype(jnp.float32)
        acc = jnp.dot(oh, t_ref[...], preferred_element_type=jnp.float32, precision=lax.Precision.HIGHEST)
        acc = acc + x[:, 1:2].astype(jnp.float32) * s_ref[...][0:1, :]
        if need_mask:
            rid = i * blk + lax.broadcasted_iota(jnp.int32, (blk, 1), 0)
            acc = jnp.where(rid < R, acc, 0.0)
        o_ref[...] = acc

    return pl.pallas_call(
        body, grid=(nb,),
        in_specs=[pl.BlockSpec((blk, 8), lambda i: (i, 0)),
                  pl.BlockSpec((32, 64), lambda i: (0, 0)),
                  pl.BlockSpec((8, 64), lambda i: (0, 0))],
        out_specs=pl.BlockSpec((blk, 64), lambda i: (i, 0)),
        out_shape=jax.ShapeDtypeStruct((Rpad, 64), jnp.float32),
    )(ftp, table, jnp.broadcast_to(svec[None, :], (8, 64)))


def _tc_fragmax(pres):
    Rpad = pres.shape[0]
    nb = Rpad // _BLK

    def body(p_ref, o_ref):
        p = p_ref[...]
        iot = lax.broadcasted_iota(jnp.int32, (1, 512), 1)
        cand = jnp.where(p > 0.0, iot, 0)
        mx = jnp.max(cand, axis=1, keepdims=True)
        o_ref[...] = jnp.broadcast_to(mx, (_BLK, 128))

    return pl.pallas_call(
        body, grid=(nb,),
        in_specs=[pl.BlockSpec((_BLK, 512), lambda i: (i, 0))],
        out_specs=pl.BlockSpec((_BLK, 128), lambda i: (i, 0)),
        out_shape=jax.ShapeDtypeStruct((Rpad, 128), jnp.int32),
    )(pres)


def _tc_readout(sa, ca, se, ce, sf, cf, Wo, bo):
    Wp = jnp.zeros((64, 8), jnp.float32).at[:, 0].set(Wo[:, 0])
    bp = jnp.zeros((8, 8), jnp.float32).at[0, 0].set(bo[0])

    def body(sa_r, ca_r, se_r, ce_r, sf_r, cf_r, w_r, b_r, o_ref):
        pooled = (sa_r[...] / jnp.maximum(ca_r[...][:, 0:1], 1.0)
                  + se_r[...] / jnp.maximum(ce_r[...][:, 0:1], 1.0)
                  + sf_r[...] / jnp.maximum(cf_r[...][:, 0:1], 1.0))
        o_ref[...] = jnp.dot(pooled, w_r[...],
                             preferred_element_type=jnp.float32) + b_r[...][0:1, :]

    full = lambda shp: pl.BlockSpec(shp, lambda: tuple(0 for _ in shp))
    return pl.pallas_call(
        body,
        in_specs=[full((512, 64)), full((512, 8)), full((512, 64)),
                  full((512, 8)), full((512, 64)), full((512, 8)),
                  full((64, 8)), full((8, 8))],
        out_specs=full((512, 8)),
        out_shape=jax.ShapeDtypeStruct((512, 8), jnp.float32),
    )(sa, ca, se, ce, sf, cf, Wp, bp)


def _pad_rows(a, rpad, val=0):
    r = a.shape[0]
    if r == rpad:
        return a
    pad = [(0, rpad - r)] + [(0, 0)] * (a.ndim - 1)
    return jnp.pad(a, pad, constant_values=val)


def _pad_idx(idx, kpad, fill):
    k = idx.shape[0]
    idx = idx.astype(jnp.int32)
    if k == kpad:
        return idx
    return jnp.concatenate([idx, jnp.full((kpad - k,), fill, jnp.int32)])


def kernel(x, edge_index, edge_attr, fragments_edge_index, batch,
           fragment_types, params):
    N = x.shape[0]
    E = edge_index.shape[1]
    NF = fragment_types.shape[0]
    AF = fragments_edge_index.shape[1]
    B = 512
    NP = _rup(N, 4096)
    EP = _rup(E, 4096)
    NFP = _rup(NF, 4096)
    AFP = _rup(AF, 4096)
    BP = 512

    x = x.astype(jnp.int32)
    edge_index = edge_index.astype(jnp.int32)
    edge_attr = edge_attr.astype(jnp.int32)
    fragments_edge_index = fragments_edge_index.astype(jnp.int32)
    batch = batch.astype(jnp.int32)
    fragment_types = fragment_types.astype(jnp.int32)

    row_e, col_e = edge_index[0], edge_index[1]
    rowf, colf = fragments_edge_index[0], fragments_edge_index[1]

    row_e_g = _pad_idx(row_e, EP, 0)
    col_e_g = _pad_idx(col_e, EP, 0)
    rowf_g = _pad_idx(rowf, AFP, 0)
    colf_g = _pad_idx(colf, AFP, 0)
    col_e_s = _pad_idx(col_e, EP, NP)
    rowf_s = _pad_idx(rowf, AFP, NP)
    colf_s = _pad_idx(colf, AFP, NFP)
    batch_s = _pad_idx(batch, NP, BP)

    if _USE_JNP_EMBED:
        h0 = jnp.zeros((N, 64), jnp.float32)
        for f in range(9):
            h0 = h0 + params["atom_tables"][f][x[:, f]]
        h = _pad_rows(h0, NP)
        he0 = jnp.zeros((E, 64), jnp.float32)
        for f in range(3):
            he0 = he0 + params["bond_tables"][f][edge_attr[:, f]]
        he_raw = _pad_rows(he0, EP)
        hf0 = (params["frag_type_table"][fragment_types[:, 0]]
               + fragment_types[:, 1].astype(jnp.float32)[:, None]
               * params["frag_size_vec"][None, :])
        hf = _pad_rows(hf0, NFP)
    else:
        xp = _pad_rows(jnp.pad(x, ((0, 0), (0, 7))), NP)
        h = _tc_embed_atoms(xp, params["atom_tables"], N)
        eap = _pad_rows(jnp.pad(edge_attr, ((0, 0), (0, 5))), EP)
        he_raw = _tc_embed_edges(eap, params["bond_tables"], E)
        ftp = _pad_rows(jnp.pad(fragment_types, ((0, 0), (0, 6))), NFP)
        hf = _tc_embed_frags(ftp, params["frag_type_table"],
                             params["frag_size_vec"], NF)
    he_stats = None

    batch_t = jnp.tile(batch[:, None], (1, 16))

    for lp in params["layers"]:
        W1 = lp["a2a"]["before"][0]
        A1, A2 = lp["a2a"]["after"]
        F1, F2 = lp["f2a"]["after"]
        E1, E2 = lp["a2e"]["after"]
        G1, G2 = lp["a2f"]["after"]
        CA = lp["comb_atom"][0]
        CEp = lp["comb_edge"][0]
        CFp = lp["comb_frag"][0]

        gh = _gather(h, row_e_g)
        he_part = {"x": he_raw, "stats": he_stats,
                   "relu": he_stats is not None}
        z1, st1 = _tc_stage([{"x": gh}, he_part], W1["W"], W1["b"], E)
        z = _tc_stage([{"x": z1, "stats": st1, "relu": True}], None, None, E,
                      want_stats=False)
        s1, c1 = _scatter(z, col_e_s, NP)
        m1a, sa = _tc_stage([{"x": s1, "cnt": c1}], A1["W"], A1["b"], N)
        m1b, sb = _tc_stage([{"x": m1a, "stats": sa, "relu": True}],
                            A2["W"], A2["b"], N)

        gu = _gather(hf, colf_g)
        s2, c2 = _scatter(gu, rowf_s, NP)
        m2a, sc = _tc_stage([{"x": s2, "cnt": c2}], F1["W"], F1["b"], N)
        m2b, sd = _tc_stage([{"x": m2a, "stats": sc, "relu": True}],
                            F2["W"], F2["b"], N)

        cc, sec = _tc_stage([{"x": m1b, "stats": sb, "relu": True},
                             {"x": m2b, "stats": sd, "relu": True}],
                            CA["W"], CA["b"], N)
        hraw, shs = _tc_stage([{"x": h},
                               {"x": cc, "stats": sec, "relu": True}],
                              None, None, N)
        h = _tc_stage([{"x": hraw, "stats": shs, "relu": True}], None, None,
                      N, want_stats=False)

        gr = _gather(h, row_e_g)
        gc = _gather(h, col_e_g)
        zm1, se1 = _tc_stage([{"x": gr, "scale": 0.5},
                              {"x": gc, "scale": 0.5}], E1["W"], E1["b"], E,
                             pre_add=True)
        zm2, se2 = _tc_stage([{"x": zm1, "stats": se1, "relu": True}],
                             E2["W"], E2["b"], E)
        zd, se3 = _tc_stage([{"x": zm2, "stats": se2, "relu": True}],
                            CEp["W"], CEp["b"], E)
        he_raw, he_stats = _tc_stage(
            [{"x": he_raw, "stats": he_stats, "relu": he_stats is not None},
             {"x": zd, "stats": se3, "relu": True}], None, None, E)

        gv = _gather(h, rowf_g)
        s3, c3 = _scatter(gv, colf_s, NFP)
        mfa, sf1 = _tc_stage([{"x": s3, "cnt": c3}], G1["W"], G1["b"], NF)
        mfb, sf2 = _tc_stage([{"x": mfa, "stats": sf1, "relu": True}],
                             G2["W"], G2["b"], NF)
        cfr, sfc = _tc_stage([{"x": mfb, "stats": sf2, "relu": True}],
                             CFp["W"], CFp["b"], NF)
        fraw, sf3 = _tc_stage([{"x": hf},
                               {"x": cfr, "stats": sfc, "relu": True}],
                              None, None, NF)
        hf = _tc_stage([{"x": fraw, "stats": sf3, "relu": True}], None, None,
                       NF, want_stats=False)

    AO1, AO2 = params["atom_out"]
    za1, sa1 = _tc_stage([{"x": h}], AO1["W"], AO1["b"], N)
    za2, sa2 = _tc_stage([{"x": za1, "stats": sa1, "relu": True}],
                         AO2["W"], AO2["b"], N)
    ha = _tc_stage([{"x": za2, "stats": sa2, "relu": True}], None, None, N,
                   want_stats=False)
    EO1, EO2 = params["edge_out"]
    ze1, sE1 = _tc_stage([{"x": he_raw, "stats": he_stats, "relu": True}],
                         EO1["W"], EO1["b"], E)
    ze2, sE2 = _tc_stage([{"x": ze1, "stats": sE1, "relu": True}],
                         EO2["W"], EO2["b"], E)
    hee = _tc_stage([{"x": ze2, "stats": sE2, "relu": True}], None, None, E,
                    want_stats=False)
    FO1, FO2 = params["frag_out"]
    zf1, sF1 = _tc_stage([{"x": hf}], FO1["W"], FO1["b"], NF)
    zf2, sF2 = _tc_stage([{"x": zf1, "stats": sF1, "relu": True}],
                         FO2["W"], FO2["b"], NF)
    hff = _tc_stage([{"x": zf2, "stats": sF2, "relu": True}], None, None, NF,
                    want_stats=False)

    bat_rowf = _gather(batch_t, rowf_g)[:, 0]
    comb = colf_g * B + bat_rowf
    NPRES = NF * B
    comb_s = jnp.where(jnp.arange(AFP) < AF, comb, NPRES).astype(jnp.int32)
    pres = _scatter(None, comb_s, NPRES)
    pres2 = _pad_rows(pres.reshape(NF, B), NFP)
    fb = _tc_fragmax(pres2)[:, 0]
    fb_s = jnp.where(jnp.arange(NFP) < NF, fb, BP).astype(jnp.int32)

    bat_row_e = _gather(batch_t, row_e_g)[:, 0]
    bre_s = jnp.where(jnp.arange(EP) < E, bat_row_e, BP).astype(jnp.int32)

    sA, cA = _scatter(ha, batch_s, BP)
    sE, cE = _scatter(hee, bre_s, BP)
    sF, cF = _scatter(hff, fb_s, BP)

    O = params["out"][0]
    out = _tc_readout(sA, cA, sE, cE, sF, cF, O["W"], O["b"])
    return out[:, 0:1]


# speedup vs baseline: 2.5130x; 1.0389x over previous
import functools

import jax
import jax.numpy as jnp
from jax import lax
from jax.experimental import pallas as pl
from jax.experimental.pallas import tpu as pltpu
from jax.experimental.pallas import tpu_sc as plsc

_NC = 2
_NS = 16
_NW = _NC * _NS
_CH = 128
_BLK = 4096
_EPS = 1e-5


def _rup(v, m):
    return (v + m - 1) // m * m


def _sc_gather(table, idx_pad):
    K = idx_pad.shape[0]
    W = table.shape[1]
    per_w = K // _NW
    n_it = per_w // _CH
    mesh = plsc.VectorSubcoreMesh(core_axis_name="c", subcore_axis_name="s")

    SUP = 4 * _CH
    n_sup = per_w // SUP
    n_rem = (per_w % SUP) // _CH

    @functools.partial(
        pl.kernel,
        out_type=jax.ShapeDtypeStruct((K, W), table.dtype),
        mesh=mesh,
        compiler_params=pltpu.CompilerParams(use_tc_tiling_on_sc=False),
        scratch_types=[
            pltpu.VMEM((SUP,), jnp.int32),
            pltpu.VMEM((SUP, W), table.dtype),
            pltpu.SemaphoreType.DMA,
        ],
    )
    def k(table_hbm, idx_hbm, out_hbm, idx_v, rows_v, sem):
        wid = lax.axis_index("s") * _NC + lax.axis_index("c")
        base = wid * per_w

        def sup_body(i, carry):
            off = base + i * SUP
            pltpu.sync_copy(idx_hbm.at[pl.ds(off, SUP)], idx_v)
            cps = [pltpu.async_copy(
                table_hbm.at[idx_v.at[pl.ds(kk * _CH, _CH)]],
                rows_v.at[pl.ds(kk * _CH, _CH)], sem) for kk in range(4)]
            for cp in cps:
                cp.wait()
            pltpu.sync_copy(rows_v, out_hbm.at[pl.ds(off, SUP)])
            return carry

        lax.fori_loop(0, n_sup, sup_body, 0)

        def rem_body(i, carry):
            off = base + n_sup * SUP + i * _CH
            pltpu.sync_copy(idx_hbm.at[pl.ds(off, _CH)],
                            idx_v.at[pl.ds(0, _CH)])
            pltpu.async_copy(table_hbm.at[idx_v.at[pl.ds(0, _CH)]],
                             rows_v.at[pl.ds(0, _CH)], sem).wait()
            pltpu.sync_copy(rows_v.at[pl.ds(0, _CH)],
                            out_hbm.at[pl.ds(off, _CH)])
            return carry

        if n_rem:
            lax.fori_loop(0, n_rem, rem_body, 0)

    return k(table, idx_pad)


def _zero_region(zsrc, dst, row0, rows):
    zr = zsrc.shape[0]
    full, rem = rows // zr, rows % zr
    for j in range(full):
        pltpu.sync_copy(zsrc.at[pl.ds(0, zr)], dst.at[pl.ds(row0 + j * zr, zr)])
    if rem:
        pltpu.sync_copy(zsrc.at[pl.ds(0, rem)],
                        dst.at[pl.ds(row0 + full * zr, rem)])


def _sc_scatter(data, idx_pad, num_out_pad, width=64):
    K = idx_pad.shape[0]
    H = num_out_pad // 2
    per_t = K // _NS
    ptr = H // _NS
    has_data = data is not None
    mesh = plsc.VectorSubcoreMesh(core_axis_name="c", subcore_axis_name="s")

    shared_words = (H + 16) * ((width + 1) if has_data else 1)
    avail = (2097151 - shared_words) // 16 - 4096
    per_row = (width + 2) if has_data else 2
    SUP = max(1, min(4, (avail // per_row) // _CH)) * _CH
    NK = SUP // _CH
    n_sup = per_t // SUP
    n_rem = (per_t % SUP) // _CH

    loc_scratch = [pltpu.VMEM((_CH,), jnp.int32) for _ in range(NK)]
    if has_data:
        out_type = (jax.ShapeDtypeStruct((2 * H, width), jnp.float32),
                    jax.ShapeDtypeStruct((2 * H,), jnp.float32))
        scratch = ([pltpu.VMEM((SUP,), jnp.int32)] + loc_scratch +
                   [pltpu.VMEM((SUP, width), jnp.float32),
                    pltpu.VMEM((_CH,), jnp.float32),
                    pltpu.VMEM_SHARED((H + 16, width), jnp.float32),
                    pltpu.VMEM_SHARED((H + 16,), jnp.float32)])
    else:
        out_type = jax.ShapeDtypeStruct((2 * H,), jnp.float32)
        scratch = ([pltpu.VMEM((SUP,), jnp.int32)] + loc_scratch +
                   [pltpu.VMEM((_CH,), jnp.float32),
                    pltpu.VMEM_SHARED((H + 16,), jnp.float32)])

    @functools.partial(pl.kernel, out_type=out_type, mesh=mesh,
                       compiler_params=pltpu.CompilerParams(
                           use_tc_tiling_on_sc=False),
                       scratch_types=scratch)
    def k(*refs):
        if has_data:
            (data_hbm, idx_hbm, ones_hbm, zw_hbm, z1_hbm,
             sums_hbm, cnt_hbm,
             idx_v, *rest) = refs
            locs = rest[:NK]
            data_v, ones_v, sp_sum, sp_cnt = rest[NK:]
        else:
            (idx_hbm, ones_hbm, z1_hbm,
             cnt_hbm,
             idx_v, *rest) = refs
            locs = rest[:NK]
            ones_v, sp_cnt = rest[NK:]
        c = lax.axis_index("c")
        s = lax.axis_index("s")
        cbase = c * H

        pltpu.sync_copy(ones_hbm, ones_v)
        if has_data:
            _zero_region(zw_hbm, sp_sum, s * ptr, ptr)
        _zero_region(z1_hbm, sp_cnt, s * ptr, ptr)
        plsc.subcore_barrier()

        base = s * per_t

        def localize(kk):
            for k8 in range(_CH // 16):
                iv = idx_v[pl.ds(kk * _CH + k8 * 16, 16)]
                loc = iv - cbase
                inb = (iv >= cbase) & (iv < cbase + H)
                locs[kk][pl.ds(k8 * 16, 16)] = jnp.where(inb, loc, H)

        def sup_body(i, carry):
            off = base + i * SUP
            pltpu.sync_copy(idx_hbm.at[pl.ds(off, SUP)], idx_v)
            if has_data:
                pltpu.sync_copy(data_hbm.at[pl.ds(off, SUP)], data_v)
            for kk in range(NK):
                localize(kk)
            for kk in range(NK):
                if has_data:
                    pltpu.sync_copy(data_v.at[pl.ds(kk * _CH, _CH)],
                                    sp_sum.at[locs[kk]], add=True)
                pltpu.sync_copy(ones_v, sp_cnt.at[locs[kk]], add=True)
            return carry

        lax.fori_loop(0, n_sup, sup_body, 0)

        def rem_body(i, carry):
            off = base + n_sup * SUP + i * _CH
            pltpu.sync_copy(idx_hbm.at[pl.ds(off, _CH)],
                            idx_v.at[pl.ds(0, _CH)])
            if has_data:
                pltpu.sync_copy(data_hbm.at[pl.ds(off, _CH)],
                                data_v.at[pl.ds(0, _CH)])
            localize(0)
            if has_data:
                pltpu.sync_copy(data_v.at[pl.ds(0, _CH)],
                                sp_sum.at[locs[0]], add=True)
            pltpu.sync_copy(ones_v, sp_cnt.at[locs[0]], add=True)
            return carry

        if n_rem:
            lax.fori_loop(0, n_rem, rem_body, 0)
        plsc.subcore_barrier()

        r0 = s * ptr
        if has_data:
            pltpu.sync_copy(sp_sum.at[pl.ds(r0, ptr)],
                            sums_hbm.at[pl.ds(cbase + r0, ptr)])
        pltpu.sync_copy(sp_cnt.at[pl.ds(r0, ptr)],
                        cnt_hbm.at[pl.ds(cbase + r0, ptr)])

    ones = jnp.ones((_CH,), jnp.float32)
    z1 = jnp.zeros((2048,), jnp.float32)
    if has_data:
        zw = jnp.zeros((2048, width), jnp.float32)
        return k(data, idx_pad, ones, zw, z1)
    return k(idx_pad, ones, z1)


_USE_SC_GATHER = True
_USE_SC_SCATTER = True
_USE_JNP_EMBED = False


def _gather(table, idx_pad):
    if _USE_SC_GATHER:
        return _sc_gather(table, idx_pad)
    return table[jnp.minimum(idx_pad, table.shape[0] - 1)]


def _t8(cnt1d):
    return jnp.tile(cnt1d[:, None], (1, 8))


def _scatter(data, idx_pad, num_out_pad, width=64):
    if _USE_SC_SCATTER:
        return _sc_scatter(data, idx_pad, num_out_pad, width)
    if data is None:
        return jax.ops.segment_sum(jnp.ones(idx_pad.shape, jnp.float32),
                                   idx_pad, num_segments=num_out_pad + 1)[:num_out_pad]
    sums = jax.ops.segment_sum(data, idx_pad,
                               num_segments=num_out_pad + 1)[:num_out_pad]
    cnt = jax.ops.segment_sum(jnp.ones(idx_pad.shape, jnp.float32), idx_pad,
                              num_segments=num_out_pad + 1)[:num_out_pad]
    return sums, cnt


def _tc_stage(parts, Wmat, bias, R, want_stats=True, blk=_BLK, pre_add=False):
    Rpad = parts[0]["x"].shape[0]
    nb = Rpad // blk
    has_w = Wmat is not None
    has_b = bias is not None
    dout = Wmat.shape[1] if has_w else parts[0]["x"].shape[1]
    cfg = tuple((p["x"].shape[1], p.get("stats") is not None, p.get("relu", False),
                 p.get("cnt") is not None, p.get("scale", 1.0)) for p in parts)
    need_mask = R < Rpad

    def body(*refs):
        it = iter(refs)
        i = pl.program_id(0)
        xs = []
        for (d, has_s, rl, has_c, scl) in cfg:
            xr = next(it)
            st = next(it) if has_s else None
            cr = next(it) if has_c else None
            x = xr[...]
            if has_s:
                sv = st[...]
                sr = p.get("stats_R", R)
                d0 = sv[0:1, :] * (1.0 / sr)
                m = sv[2:3, :] + d0
                var = jnp.maximum(sv[1:2, :] * (1.0 / sr) - d0 * d0, 0.0)
                x = (x - m) / jnp.sqrt(var + _EPS)
            if rl:
                x = jnp.maximum(x, 0.0)
            if has_c:
                cnt = cr[...][:, 0:1]
                x = x / jnp.maximum(cnt, 1.0)
            if scl != 1.0:
                x = x * scl
            xs.append(x)
        if has_w:
            wr = next(it)
            if pre_add:
                x = xs[0]
                for xo in xs[1:]:
                    x = x + xo
                acc = jnp.dot(x, wr[...],
                              preferred_element_type=jnp.float32)
            else:
                acc = None
                woff = 0
                for x, (d, *_rest) in zip(xs, cfg):
                    t = jnp.dot(x, wr[pl.ds(woff, d), :],
                                preferred_element_type=jnp.float32)
                    acc = t if acc is None else acc + t
                    woff += d
        else:
            acc = xs[0]
            for x in xs[1:]:
                acc = acc + x
        if has_b:
            acc = acc + next(it)[...][0:1, :]
        if need_mask:
            rid = i * blk + lax.broadcasted_iota(jnp.int32, (blk, 1), 0)
            acc = jnp.where(rid < R, acc, 0.0)
        z_ref = next(it)
        z_ref[...] = acc
        if want_stats:
            st_ref = next(it)

            @pl.when(i == 0)
            def _():
                st_ref[...] = jnp.zeros((8, dout), jnp.float32)
                st_ref[2:3, :] = jnp.sum(acc, axis=0, keepdims=True) * (1.0 / blk)

            cshift = st_ref[2:3, :]
            d = acc - cshift
            if need_mask:
                rid2 = i * blk + lax.broadcasted_iota(jnp.int32, (blk, 1), 0)
                d = jnp.where(rid2 < R, d, 0.0)
            st_ref[0:1, :] += jnp.sum(d, axis=0, keepdims=True)
            st_ref[1:2, :] += jnp.sum(d * d, axis=0, keepdims=True)

    in_specs = []
    args = []
    for p, (d, has_s, rl, has_c, scl) in zip(parts, cfg):
        in_specs.append(pl.BlockSpec((blk, d), lambda i: (i, 0)))
        args.append(p["x"])
        if has_s:
            in_specs.append(pl.BlockSpec((8, d), lambda i: (0, 0)))
            args.append(p["stats"])
        if has_c:
            in_specs.append(pl.BlockSpec((blk, 8), lambda i: (i, 0)))
            args.append(p["cnt"])
    if has_w:
        dtot = Wmat.shape[0]
        in_specs.append(pl.BlockSpec((dtot, dout), lambda i: (0, 0)))
        args.append(Wmat)
    if has_b:
        in_specs.append(pl.BlockSpec((8, dout), lambda i: (0, 0)))
        args.append(jnp.broadcast_to(bias[None, :], (8, dout)))

    out_shape = [jax.ShapeDtypeStruct((Rpad, dout), jnp.float32)]
    out_specs = [pl.BlockSpec((blk, dout), lambda i: (i, 0))]
    if want_stats:
        out_shape.append(jax.ShapeDtypeStruct((8, dout), jnp.float32))
        out_specs.append(pl.BlockSpec((8, dout), lambda i: (0, 0)))

    res = pl.pallas_call(
        body, grid=(nb,), in_specs=in_specs, out_specs=out_specs,
        out_shape=out_shape,
    )(*args)
    return (res[0], res[1]) if want_stats else res[0]


def _tc_embed_atoms(xp, tables, R):
    Rpad = xp.shape[0]
    nb = Rpad // _BLK
    need_mask = R < Rpad

    def body(x_ref, t_ref, o_ref):
        i = pl.program_id(0)
        x = x_ref[...]
        acc = jnp.zeros((_BLK, 64), jnp.float32)
        iot = lax.broadcasted_iota(jnp.int32, (1, 128), 1)
        for f in range(9):
            oh = (x[:, f:f + 1] == iot).astype(jnp.float32)
            acc = acc + jnp.dot(oh, t_ref[f], preferred_element_type=jnp.float32, precision=lax.Precision.HIGHEST)
        if need_mask:
            rid = i * _BLK + lax.broadcasted_iota(jnp.int32, (_BLK, 1), 0)
            acc = jnp.where(rid < R, acc, 0.0)
        o_ref[...] = acc

    return pl.pallas_call(
        body, grid=(nb,),
        in_specs=[pl.BlockSpec((_BLK, 16), lambda i: (i, 0)),
                  pl.BlockSpec((9, 128, 64), lambda i: (0, 0, 0))],
        out_specs=pl.BlockSpec((_BLK, 64), lambda i: (i, 0)),
        out_shape=jax.ShapeDtypeStruct((Rpad, 64), jnp.float32),
    )(xp, tables)


def _tc_embed_edges(eap, tables, R):
    Rpad = eap.shape[0]
    nb = Rpad // _BLK
    need_mask = R < Rpad

    def body(x_ref, t_ref, o_ref):
        i = pl.program_id(0)
        x = x_ref[...]
        acc = jnp.zeros((_BLK, 64), jnp.float32)
        iot = lax.broadcasted_iota(jnp.int32, (1, 32), 1)
        for f in range(3):
            oh = (x[:, f:f + 1] == iot).astype(jnp.float32)
            acc = acc + jnp.dot(oh, t_ref[f], preferred_element_type=jnp.float32, precision=lax.Precision.HIGHEST)
        if need_mask:
            rid = i * _BLK + lax.broadcasted_iota(jnp.int32, (_BLK, 1), 0)
            acc = jnp.where(rid < R, acc, 0.0)
        o_ref[...] = acc

    return pl.pallas_call(
        body, grid=(nb,),
        in_specs=[pl.BlockSpec((_BLK, 8), lambda i: (i, 0)),
                  pl.BlockSpec((3, 32, 64), lambda i: (0, 0, 0))],
        out_specs=pl.BlockSpec((_BLK, 64), lambda i: (i, 0)),
        out_shape=jax.ShapeDtypeStruct((Rpad, 64), jnp.float32),
    )(eap, tables)


def _tc_embed_frags(ftp, table, svec, R):
    Rpad = ftp.shape[0]
    nb = Rpad // _BLK if Rpad >= _BLK else 1
    blk = Rpad // nb
    need_mask = R < Rpad

    def body(x_ref, t_ref, s_ref, o_ref):
        i = pl.program_id(0)
        x = x_ref[...]
        iot = lax.broadcasted_iota(jnp.int32, (1, 32), 1)
        oh = (x[:, 0:1] == iot).astype(jnp.float32)
        acc = jnp.dot(oh, t_ref[...], preferred_element_type=jnp.float32, precision=lax.Precision.HIGHEST)
        acc = acc + x[:, 1:2].astype(jnp.float32) * s_ref[...][0:1, :]
        if need_mask:
            rid = i * blk + lax.broadcasted_iota(jnp.int32, (blk, 1), 0)
            acc = jnp.where(rid < R, acc, 0.0)
        o_ref[...] = acc

    return pl.pallas_call(
        body, grid=(nb,),
        in_specs=[pl.BlockSpec((blk, 8), lambda i: (i, 0)),
                  pl.BlockSpec((32, 64), lambda i: (0, 0)),
                  pl.BlockSpec((8, 64), lambda i: (0, 0))],
        out_specs=pl.BlockSpec((blk, 64), lambda i: (i, 0)),
        out_shape=jax.ShapeDtypeStruct((Rpad, 64), jnp.float32),
    )(ftp, table, jnp.broadcast_to(svec[None, :], (8, 64)))


def _tc_fragmax(pres):
    Rpad = pres.shape[0]
    nb = Rpad // _BLK

    def body(p_ref, o_ref):
        p = p_ref[...]
        iot = lax.broadcasted_iota(jnp.int32, (1, 512), 1)
        cand = jnp.where(p > 0.0, iot, 0)
        mx = jnp.max(cand, axis=1, keepdims=True)
        o_ref[...] = jnp.broadcast_to(mx, (_BLK, 128))

    return pl.pallas_call(
        body, grid=(nb,),
        in_specs=[pl.BlockSpec((_BLK, 512), lambda i: (i, 0))],
        out_specs=pl.BlockSpec((_BLK, 128), lambda i: (i, 0)),
        out_shape=jax.ShapeDtypeStruct((Rpad, 128), jnp.int32),
    )(pres)


def _tc_readout(sa, ca, se, ce, sf, cf, Wo, bo):
    Wp = jnp.zeros((64, 8), jnp.float32).at[:, 0].set(Wo[:, 0])
    bp = jnp.zeros((8, 8), jnp.float32).at[0, 0].set(bo[0])

    def body(sa_r, ca_r, se_r, ce_r, sf_r, cf_r, w_r, b_r, o_ref):
        pooled = (sa_r[...] / jnp.maximum(ca_r[...][:, 0:1], 1.0)
                  + se_r[...] / jnp.maximum(ce_r[...][:, 0:1], 1.0)
                  + sf_r[...] / jnp.maximum(cf_r[...][:, 0:1], 1.0))
        o_ref[...] = jnp.dot(pooled, w_r[...],
                             preferred_element_type=jnp.float32) + b_r[...][0:1, :]

    full = lambda shp: pl.BlockSpec(shp, lambda: tuple(0 for _ in shp))
    return pl.pallas_call(
        body,
        in_specs=[full((512, 64)), full((512, 8)), full((512, 64)),
                  full((512, 8)), full((512, 64)), full((512, 8)),
                  full((64, 8)), full((8, 8))],
        out_specs=full((512, 8)),
        out_shape=jax.ShapeDtypeStruct((512, 8), jnp.float32),
    )(sa, ca, se, ce, sf, cf, Wp, bp)


def _pad_rows(a, rpad, val=0):
    r = a.shape[0]
    if r == rpad:
        return a
    pad = [(0, rpad - r)] + [(0, 0)] * (a.ndim - 1)
    return jnp.pad(a, pad, constant_values=val)


def _pad_idx(idx, kpad, fill):
    k = idx.shape[0]
    idx = idx.astype(jnp.int32)
    if k == kpad:
        return idx
    return jnp.concatenate([idx, jnp.full((kpad - k,), fill, jnp.int32)])


def kernel(x, edge_index, edge_attr, fragments_edge_index, batch,
           fragment_types, params):
    N = x.shape[0]
    E = edge_index.shape[1]
    NF = fragment_types.shape[0]
    AF = fragments_edge_index.shape[1]
    B = 512
    NP = _rup(N, 4096)
    EP = _rup(E, 4096)
    NFP = _rup(NF, 4096)
    AFP = _rup(AF, 4096)
    BP = 512

    x = x.astype(jnp.int32)
    edge_index = edge_index.astype(jnp.int32)
    edge_attr = edge_attr.astype(jnp.int32)
    fragments_edge_index = fragments_edge_index.astype(jnp.int32)
    batch = batch.astype(jnp.int32)
    fragment_types = fragment_types.astype(jnp.int32)

    row_e, col_e = edge_index[0], edge_index[1]
    rowf, colf = fragments_edge_index[0], fragments_edge_index[1]

    row_e_g = _pad_idx(row_e, EP, 0)
    col_e_g = _pad_idx(col_e, EP, 0)
    rowf_g = _pad_idx(rowf, AFP, 0)
    colf_g = _pad_idx(colf, AFP, 0)
    col_e_s = _pad_idx(col_e, EP, NP)
    rowf_s = _pad_idx(rowf, AFP, NP)
    colf_s = _pad_idx(colf, AFP, NFP)
    batch_s = _pad_idx(batch, NP, BP)

    if _USE_JNP_EMBED:
        h0 = jnp.zeros((N, 64), jnp.float32)
        for f in range(9):
            h0 = h0 + params["atom_tables"][f][x[:, f]]
        h = _pad_rows(h0, NP)
        he0 = jnp.zeros((E, 64), jnp.float32)
        for f in range(3):
            he0 = he0 + params["bond_tables"][f][edge_attr[:, f]]
        he_raw = _pad_rows(he0, EP)
        hf0 = (params["frag_type_table"][fragment_types[:, 0]]
               + fragment_types[:, 1].astype(jnp.float32)[:, None]
               * params["frag_size_vec"][None, :])
        hf = _pad_rows(hf0, NFP)
    else:
        xp = _pad_rows(jnp.pad(x, ((0, 0), (0, 7))), NP)
        h = _tc_embed_atoms(xp, params["atom_tables"], N)
        eap = _pad_rows(jnp.pad(edge_attr, ((0, 0), (0, 5))), EP)
        he_raw = _tc_embed_edges(eap, params["bond_tables"], E)
        ftp = _pad_rows(jnp.pad(fragment_types, ((0, 0), (0, 6))), NFP)
        hf = _tc_embed_frags(ftp, params["frag_type_table"],
                             params["frag_size_vec"], NF)
    he_stats = None

    batch_t = jnp.tile(batch[:, None], (1, 16))

    for lp in params["layers"]:
        W1 = lp["a2a"]["before"][0]
        A1, A2 = lp["a2a"]["after"]
        F1, F2 = lp["f2a"]["after"]
        E1, E2 = lp["a2e"]["after"]
        G1, G2 = lp["a2f"]["after"]
        CA = lp["comb_atom"][0]
        CEp = lp["comb_edge"][0]
        CFp = lp["comb_frag"][0]

        gh = _gather(h, row_e_g)
        he_part = {"x": he_raw, "stats": he_stats,
                   "relu": he_stats is not None}
        z1, st1 = _tc_stage([{"x": gh}, he_part], W1["W"], W1["b"], E)
        z = _tc_stage([{"x": z1, "stats": st1, "relu": True}], None, None, E,
                      want_stats=False)
        s1, c1 = _scatter(z, col_e_s, NP)
        m1a, sa = _tc_stage([{"x": s1, "cnt": _t8(c1)}], A1["W"], A1["b"], N)
        m1b, sb = _tc_stage([{"x": m1a, "stats": sa, "relu": True}],
                            A2["W"], A2["b"], N)

        gu = _gather(hf, colf_g)
        s2, c2 = _scatter(gu, rowf_s, NP)
        m2a, sc = _tc_stage([{"x": s2, "cnt": _t8(c2)}], F1["W"], F1["b"], N)
        m2b, sd = _tc_stage([{"x": m2a, "stats": sc, "relu": True}],
                            F2["W"], F2["b"], N)

        cc, sec = _tc_stage([{"x": m1b, "stats": sb, "relu": True},
                             {"x": m2b, "stats": sd, "relu": True}],
                            CA["W"], CA["b"], N)
        hraw, shs = _tc_stage([{"x": h},
                               {"x": cc, "stats": sec, "relu": True}],
                              None, None, N)
        h = _tc_stage([{"x": hraw, "stats": shs, "relu": True}], None, None,
                      N, want_stats=False)

        gr = _gather(h, row_e_g)
        gc = _gather(h, col_e_g)
        zm1, se1 = _tc_stage([{"x": gr, "scale": 0.5},
                              {"x": gc, "scale": 0.5}], E1["W"], E1["b"], E,
                             pre_add=True)
        zm2, se2 = _tc_stage([{"x": zm1, "stats": se1, "relu": True}],
                             E2["W"], E2["b"], E)
        zd, se3 = _tc_stage([{"x": zm2, "stats": se2, "relu": True}],
                            CEp["W"], CEp["b"], E)
        he_raw, he_stats = _tc_stage(
            [{"x": he_raw, "stats": he_stats, "relu": he_stats is not None},
             {"x": zd, "stats": se3, "relu": True}], None, None, E)

        gv = _gather(h, rowf_g)
        s3, c3 = _scatter(gv, colf_s, NFP)
        mfa, sf1 = _tc_stage([{"x": s3, "cnt": _t8(c3)}], G1["W"], G1["b"], NF)
        mfb, sf2 = _tc_stage([{"x": mfa, "stats": sf1, "relu": True}],
                             G2["W"], G2["b"], NF)
        cfr, sfc = _tc_stage([{"x": mfb, "stats": sf2, "relu": True}],
                             CFp["W"], CFp["b"], NF)
        fraw, sf3 = _tc_stage([{"x": hf},
                               {"x": cfr, "stats": sfc, "relu": True}],
                              None, None, NF)
        hf = _tc_stage([{"x": fraw, "stats": sf3, "relu": True}], None, None,
                       NF, want_stats=False)

    AO1, AO2 = params["atom_out"]
    za1, sa1 = _tc_stage([{"x": h}], AO1["W"], AO1["b"], N)
    za2, sa2 = _tc_stage([{"x": za1, "stats": sa1, "relu": True}],
                         AO2["W"], AO2["b"], N)
    ha = _tc_stage([{"x": za2, "stats": sa2, "relu": True}], None, None, N,
                   want_stats=False)
    EO1, EO2 = params["edge_out"]
    ze1, sE1 = _tc_stage([{"x": he_raw, "stats": he_stats, "relu": True}],
                         EO1["W"], EO1["b"], E)
    ze2, sE2 = _tc_stage([{"x": ze1, "stats": sE1, "relu": True}],
                         EO2["W"], EO2["b"], E)
    hee = _tc_stage([{"x": ze2, "stats": sE2, "relu": True}], None, None, E,
                    want_stats=False)
    FO1, FO2 = params["frag_out"]
    zf1, sF1 = _tc_stage([{"x": hf}], FO1["W"], FO1["b"], NF)
    zf2, sF2 = _tc_stage([{"x": zf1, "stats": sF1, "relu": True}],
                         FO2["W"], FO2["b"], NF)
    hff = _tc_stage([{"x": zf2, "stats": sF2, "relu": True}], None, None, NF,
                    want_stats=False)

    bat_rowf = _gather(batch_t, rowf_g)[:, 0]
    comb = colf_g * B + bat_rowf
    NPRES = NF * B
    comb_s = jnp.where(jnp.arange(AFP) < AF, comb, NPRES).astype(jnp.int32)
    pres = _scatter(None, comb_s, NPRES)
    pres2 = _pad_rows(pres.reshape(NF, B), NFP)
    fb = _tc_fragmax(pres2)[:, 0]
    fb_s = jnp.where(jnp.arange(NFP) < NF, fb, BP).astype(jnp.int32)

    bat_row_e = _gather(batch_t, row_e_g)[:, 0]
    bre_s = jnp.where(jnp.arange(EP) < E, bat_row_e, BP).astype(jnp.int32)

    sA, cA = _scatter(ha, batch_s, BP)
    sE, cE = _scatter(hee, bre_s, BP)
    sF, cF = _scatter(hff, fb_s, BP)

    O = params["out"][0]
    out = _tc_readout(sA, _t8(cA), sE, _t8(cE), sF, _t8(cF), O["W"], O["b"])
    return out[:, 0:1]
